# baseline (speedup 1.0000x reference)
import jax
import jax.numpy as jnp
from jax import lax
from jax.experimental import pallas as pl
from jax.experimental.pallas import tpu as pltpu
from jax.experimental.pallas import tpu_sc as plsc

B = 1024
V = 100000
NCORE = 2
NSUB = 16
NW = NCORE * NSUB
RPW = B // NW
NRB = RPW // 8
CW = 1408
NCHUNK = 71
TAILC = 99968
NT = CW // 128
U = 8
GROUPS = NT
TOT = NRB * NCHUNK


def _merge_top16(T, v):
    vd, _ = plsc.sort_key_val(v, v, descending=True)
    m = jnp.maximum(T, vd)
    Ts, _ = plsc.sort_key_val(m, m, descending=False)
    return Ts


def _any_gt(v, t):
    return plsc.all_reduce_population_count(v > t)[0] > 0


def _scan_row(buf, i, T):
    def g_body(g, carry):
        T, tmin = carry
        vs = [buf[g, i, pl.ds(k * 16, 16)] for k in range(U)]
        w = vs[0]
        for k in range(1, U):
            w = jnp.maximum(w, vs[k])

        def do_insert(carry):
            T, tmin = carry
            for k in range(U):
                def ins(T, v=vs[k]):
                    return _merge_top16(T, v)
                T = lax.cond(_any_gt(vs[k], T[0]), ins, lambda T: T, T)
            return (T, T[0])

        return lax.cond(_any_gt(w, tmin), do_insert, lambda c: c, (T, tmin))

    T, _ = lax.fori_loop(0, GROUPS, g_body, (T, T[0]), unroll=2)
    return T


def _ysplat(ybuf, rl):
    half = jnp.full((16,), rl // 16, jnp.int32)
    yva = ybuf[pl.ds(0, 16)]
    yvb = ybuf[pl.ds(16, 16)]
    yh = jnp.where(half == 0, yva, yvb)
    return jnp.take(yh, jnp.full((16,), rl % 16, jnp.int32))


def _track_gt(buf3, ybuf, gtbuf, rl, i, c0):
    lane0 = lax.iota(jnp.int32, 16) == 0
    o = _ysplat(ybuf, rl) - c0
    inr = (o >= 0) & (o < CW)
    oc = jnp.clip(o, 0, CW - 1)
    gat = plsc.load_gather(
        buf3, [oc // 128, jnp.full((16,), i, jnp.int32), oc % 128])
    plsc.store_scatter(gtbuf, [jnp.full((16,), rl, jnp.int32)], gat,
                       mask=lane0 & inr)


def _track_gt_tail(tbuf, ybuf, gtbuf, rl, i):
    lane0 = lax.iota(jnp.int32, 16) == 0
    o = _ysplat(ybuf, rl) - TAILC
    inr = (o >= 0) & (o < 32)
    oc = jnp.clip(o, 0, 31)
    gat = plsc.load_gather(tbuf, [jnp.full((16,), i, jnp.int32), oc])
    plsc.store_scatter(gtbuf, [jnp.full((16,), rl, jnp.int32)], gat,
                       mask=lane0 & inr)


def _sc_body(x2, y, gt_out, tk_out,
             ybuf, gtbuf, tkbuf, buf0, buf1, tbuf, sem0, sem1, semt):
    c = lax.axis_index("c")
    s = lax.axis_index("s")
    wid = s * NCORE + c
    row0 = wid * RPW

    pltpu.sync_copy(y.at[pl.ds(row0, RPW)], ybuf)

    bufs = (buf0, buf1)
    sems = (sem0, sem1)

    def dma_tiles(step, b):
        rb = step // NCHUNK
        ci = step % NCHUNK
        r8 = row0 + rb * 8
        return [pltpu.make_async_copy(
                    x2.at[pl.ds(r8, 8), pl.ds((ci * NT + t) * 128, 128)],
                    bufs[b].at[t], sems[b])
                for t in range(NT)]

    for d in dma_tiles(0, 0):
        d.start()

    neg = jnp.full((16,), -jnp.inf, jnp.float32)

    def pair_body(p, carry):
        for j2 in range(2):
            step = p * 2 + j2
            b = j2
            nb = (j2 + 1) % 2

            @pl.when(step + 1 < TOT)
            def _(step=step, nb=nb):
                for d in dma_tiles(step + 1, nb):
                    d.start()

            for d in dma_tiles(step, b):
                d.wait()
            rb = step // NCHUNK
            ci = step % NCHUNK
            c0 = ci * CW
            for i in range(8):
                rl = rb * 8 + i
                T = jnp.where(ci == 0, neg, tkbuf[rl, :])
                T = _scan_row(bufs[b], i, T)
                tkbuf[rl, :] = T
                _track_gt(bufs[b], ybuf, gtbuf, rl, i, c0)
        return carry

    lax.fori_loop(0, TOT // 2, pair_body, 0)

    def tail_body(rb, carry):
        pltpu.make_async_copy(
            x2.at[pl.ds((row0 + rb * 8), 8), pl.ds(TAILC, 32)],
            tbuf, semt).start()
        pltpu.make_async_copy(
            x2.at[pl.ds((row0 + rb * 8), 8), pl.ds(TAILC, 32)],
            tbuf, semt).wait()
        for i in range(8):
            rl = rb * 8 + i
            T = tkbuf[rl, :]
            T = _merge_top16(T, tbuf[i, pl.ds(0, 16)])
            T = _merge_top16(T, tbuf[i, pl.ds(16, 16)])
            tkbuf[rl, :] = T
            _track_gt_tail(tbuf, ybuf, gtbuf, rl, i)
        return carry

    lax.fori_loop(0, NRB, tail_body, 0)

    pltpu.sync_copy(gtbuf, gt_out.at[pl.ds(row0, RPW)])
    pltpu.sync_copy(tkbuf, tk_out.at[pl.ds(row0, RPW)])


def _tc_body(gt_ref, tk_ref, out_ref):
    g = gt_ref[:, :]
    t = tk_ref[:, :]
    m = jnp.maximum(t[:, 15:16], g)
    s16 = jnp.sum(jnp.exp(t - m), axis=1, keepdims=True)
    v16 = t[:, 0:1]
    removed = jnp.where(g >= v16, g, v16)
    s15 = s16 - jnp.exp(removed - m)
    lse = m + jnp.log(s15 + jnp.exp(g - m))
    out_ref[:, :] = jnp.broadcast_to(jnp.mean(lse - g), (1, 1))


def kernel(x, y):
    yi = y.astype(jnp.int32)
    sc = pl.kernel(
        _sc_body,
        out_type=(jax.ShapeDtypeStruct((B,), jnp.float32),
                  jax.ShapeDtypeStruct((B, 16), jnp.float32)),
        mesh=plsc.VectorSubcoreMesh(core_axis_name="c", subcore_axis_name="s",
                                    num_cores=NCORE, num_subcores=NSUB),
        scratch_types=[
            pltpu.VMEM((RPW,), jnp.int32),
            pltpu.VMEM((RPW,), jnp.float32),
            pltpu.VMEM((RPW, 16), jnp.float32),
            pltpu.VMEM((NT, 8, 128), jnp.float32),
            pltpu.VMEM((NT, 8, 128), jnp.float32),
            pltpu.VMEM((8, 32), jnp.float32),
            pltpu.SemaphoreType.DMA,
            pltpu.SemaphoreType.DMA,
            pltpu.SemaphoreType.DMA,
        ],
        compiler_params=pltpu.CompilerParams(needs_layout_passes=False),
    )
    gt, tk = sc(x, yi)
    loss = pl.pallas_call(
        _tc_body,
        out_shape=jax.ShapeDtypeStruct((1, 1), jnp.float32),
    )(gt.reshape(B, 1), tk)
    return loss[0, 0]


# speedup vs baseline: 1.6208x; 1.6208x over previous
import jax
import jax.numpy as jnp
from jax import lax
from jax.experimental import pallas as pl
from jax.experimental.pallas import tpu as pltpu
from jax.experimental.pallas import tpu_sc as plsc

B = 1024
V = 100000
NCORE = 2
NSUB = 16
NW = NCORE * NSUB
RPW = B // NW
NRB = RPW // 8
CW = 1408
NCHUNK = 71
TAILC = 99968
NT = CW // 128
U = 8
GROUPS = NT
TOT = NRB * NCHUNK


def _merge_top16(T, v):
    vd, _ = plsc.sort_key_val(v, v, descending=True)
    m = jnp.maximum(T, vd)
    Ts, _ = plsc.sort_key_val(m, m, descending=False)
    return Ts


def _any_gt(v, t):
    return plsc.all_reduce_population_count(v > t)[0] > 0


def _scan_row(buf, i, T):
    def g_body(g, carry):
        T, tmin = carry
        vs = [buf[g, i, pl.ds(k * 16, 16)] for k in range(U)]
        w = vs[0]
        for k in range(1, U):
            w = jnp.maximum(w, vs[k])

        def do_insert(carry):
            T, tmin = carry

            def k_body(k, T):
                v = buf[g, i, pl.ds(k * 16, 16)]

                def ins(T):
                    return _merge_top16(T, v)

                return lax.cond(_any_gt(v, T[0]), ins, lambda T: T, T)

            T = lax.fori_loop(0, U, k_body, T)
            return (T, T[0])

        return lax.cond(_any_gt(w, tmin), do_insert, lambda c: c, (T, tmin))

    T, _ = lax.fori_loop(0, GROUPS, g_body, (T, T[0]))
    return T


def _ysplat(ybuf, rl):
    half = jnp.full((16,), rl // 16, jnp.int32)
    yva = ybuf[pl.ds(0, 16)]
    yvb = ybuf[pl.ds(16, 16)]
    yh = jnp.where(half == 0, yva, yvb)
    return jnp.take(yh, jnp.full((16,), rl % 16, jnp.int32))


def _track_gt(buf3, ybuf, gtbuf, rl, i, c0):
    lane0 = lax.iota(jnp.int32, 16) == 0
    o = _ysplat(ybuf, rl) - c0
    inr = (o >= 0) & (o < CW)
    oc = jnp.clip(o, 0, CW - 1)
    gat = plsc.load_gather(
        buf3, [oc // 128, jnp.full((16,), i, jnp.int32), oc % 128])
    plsc.store_scatter(gtbuf, [jnp.full((16,), rl, jnp.int32)], gat,
                       mask=lane0 & inr)


def _track_gt_tail(tbuf, ybuf, gtbuf, rl, i):
    lane0 = lax.iota(jnp.int32, 16) == 0
    o = _ysplat(ybuf, rl) - TAILC
    inr = (o >= 0) & (o < 32)
    oc = jnp.clip(o, 0, 31)
    gat = plsc.load_gather(tbuf, [jnp.full((16,), i, jnp.int32), oc])
    plsc.store_scatter(gtbuf, [jnp.full((16,), rl, jnp.int32)], gat,
                       mask=lane0 & inr)


def _sc_body(x2, y, gt_out, tk_out,
             ybuf, gtbuf, tkbuf, buf0, buf1, tbuf, sem0, sem1, semt):
    c = lax.axis_index("c")
    s = lax.axis_index("s")
    wid = s * NCORE + c
    row0 = wid * RPW

    pltpu.sync_copy(y.at[pl.ds(row0, RPW)], ybuf)

    bufs = (buf0, buf1)
    sems = (sem0, sem1)

    def dma_tiles(step, b):
        rb = step // NCHUNK
        ci = step % NCHUNK
        r8 = row0 + rb * 8
        return [pltpu.make_async_copy(
                    x2.at[pl.ds(r8, 8), pl.ds((ci * NT + t) * 128, 128)],
                    bufs[b].at[t], sems[b])
                for t in range(NT)]

    for d in dma_tiles(0, 0):
        d.start()

    neg = jnp.full((16,), -jnp.inf, jnp.float32)

    def pair_body(p, carry):
        for j2 in range(2):
            step = p * 2 + j2
            b = j2
            nb = (j2 + 1) % 2

            @pl.when(step + 1 < TOT)
            def _(step=step, nb=nb):
                for d in dma_tiles(step + 1, nb):
                    d.start()

            for d in dma_tiles(step, b):
                d.wait()
            rb = step // NCHUNK
            ci = step % NCHUNK
            c0 = ci * CW

            def row_body(i, carry, b=b):
                rl = rb * 8 + i
                T = jnp.where(ci == 0, neg, tkbuf[rl, :])
                T = _scan_row(bufs[b], i, T)
                tkbuf[rl, :] = T
                _track_gt(bufs[b], ybuf, gtbuf, rl, i, c0)
                return carry

            lax.fori_loop(0, 8, row_body, 0)
        return carry

    lax.fori_loop(0, TOT // 2, pair_body, 0)

    def tail_body(rb, carry):
        pltpu.make_async_copy(
            x2.at[pl.ds((row0 + rb * 8), 8), pl.ds(TAILC, 32)],
            tbuf, semt).start()
        pltpu.make_async_copy(
            x2.at[pl.ds((row0 + rb * 8), 8), pl.ds(TAILC, 32)],
            tbuf, semt).wait()
        for i in range(8):
            rl = rb * 8 + i
            T = tkbuf[rl, :]
            T = _merge_top16(T, tbuf[i, pl.ds(0, 16)])
            T = _merge_top16(T, tbuf[i, pl.ds(16, 16)])
            tkbuf[rl, :] = T
            _track_gt_tail(tbuf, ybuf, gtbuf, rl, i)
        return carry

    lax.fori_loop(0, NRB, tail_body, 0)

    pltpu.sync_copy(gtbuf, gt_out.at[pl.ds(row0, RPW)])
    pltpu.sync_copy(tkbuf, tk_out.at[pl.ds(row0, RPW)])


def _tc_body(gt_ref, tk_ref, out_ref):
    g = gt_ref[:, :]
    t = tk_ref[:, :]
    m = jnp.maximum(t[:, 15:16], g)
    s16 = jnp.sum(jnp.exp(t - m), axis=1, keepdims=True)
    v16 = t[:, 0:1]
    removed = jnp.where(g >= v16, g, v16)
    s15 = s16 - jnp.exp(removed - m)
    lse = m + jnp.log(s15 + jnp.exp(g - m))
    out_ref[:, :] = jnp.broadcast_to(jnp.mean(lse - g), (1, 1))


def kernel(x, y):
    yi = y.astype(jnp.int32)
    sc = pl.kernel(
        _sc_body,
        out_type=(jax.ShapeDtypeStruct((B,), jnp.float32),
                  jax.ShapeDtypeStruct((B, 16), jnp.float32)),
        mesh=plsc.VectorSubcoreMesh(core_axis_name="c", subcore_axis_name="s",
                                    num_cores=NCORE, num_subcores=NSUB),
        scratch_types=[
            pltpu.VMEM((RPW,), jnp.int32),
            pltpu.VMEM((RPW,), jnp.float32),
            pltpu.VMEM((RPW, 16), jnp.float32),
            pltpu.VMEM((NT, 8, 128), jnp.float32),
            pltpu.VMEM((NT, 8, 128), jnp.float32),
            pltpu.VMEM((8, 32), jnp.float32),
            pltpu.SemaphoreType.DMA,
            pltpu.SemaphoreType.DMA,
            pltpu.SemaphoreType.DMA,
        ],
        compiler_params=pltpu.CompilerParams(needs_layout_passes=False),
    )
    gt, tk = sc(x, yi)
    loss = pl.pallas_call(
        _tc_body,
        out_shape=jax.ShapeDtypeStruct((1, 1), jnp.float32),
    )(gt.reshape(B, 1), tk)
    return loss[0, 0]
